# baseline (speedup 1.0000x reference)
import functools

import jax
import jax.numpy as jnp
from jax import lax
from jax.experimental import pallas as pl
from jax.experimental.pallas import tpu as pltpu
from jax.experimental.pallas import tpu_sc as plsc

NUM_ACTIONS = 1000000
N_EMBED = 32
BATCH = 16384
HIST = 50

NC, NS = 2, 16
NW = NC * NS
N_PER_W = BATCH // NW
NB = 128
NBLK = N_PER_W // NB
NPAIR = HIST // 2
NCOLS = 7813

_mesh = plsc.VectorSubcoreMesh(core_axis_name="c", subcore_axis_name="s")


@functools.partial(
    pl.kernel,
    mesh=_mesh,
    out_type=jax.ShapeDtypeStruct((NUM_ACTIONS // 4, 128), jnp.float32),
    scratch_types=[
        pltpu.VMEM((32, 128), jnp.float32),
        pltpu.VMEM((32, 128), jnp.float32),
        pltpu.VMEM((32, 128), jnp.float32),
        pltpu.VMEM((32, 128), jnp.float32),
        pltpu.SemaphoreType.DMA,
        pltpu.SemaphoreType.DMA,
        pltpu.SemaphoreType.DMA,
        pltpu.SemaphoreType.DMA,
    ],
    compiler_params=pltpu.CompilerParams(
        use_tc_tiling_on_sc=True, needs_layout_passes=False),
)
def _detile_kernel(ttn_hbm, tail4_hbm, tab4_hbm, s0, s1, t0, t1,
                   isem0, isem1, osem0, osem1):
    wid = lax.axis_index("s") * NC + lax.axis_index("c")
    lanes = lax.iota(jnp.int32, 16)
    row_hi = lanes + 16
    bufs = ((s0, t0, isem0, osem0), (s1, t1, isem1, osem1))
    NFULL = NCOLS - 1
    ntrips = (NFULL - wid + NW - 1) // NW

    def transpose(s_v, t_v, nrows):
        def i_body(i, carry):
            for gh in range(4):
                col = jnp.full((16,), 4 * i + gh, jnp.int32)
                t_v[i, pl.ds(32 * gh, 16)] = plsc.load_gather(
                    s_v, [lanes, col])
                t_v[i, pl.ds(32 * gh + 16, 16)] = plsc.load_gather(
                    s_v, [row_hi, col])
            return carry

        lax.fori_loop(0, nrows, i_body, 0)

    pltpu.async_copy(ttn_hbm.at[:, pl.ds(wid * 128, 128)], s0, isem0)
    pltpu.async_copy(
        ttn_hbm.at[:, pl.ds((wid + NW) * 128, 128)], s1, isem1)

    def trip_pair(j, carry):
        for b, (s_v, t_v, isem, osem) in enumerate(bufs):
            t = 2 * j + b

            @pl.when(t < ntrips)
            def _do():
                c = wid + t * NW
                pltpu.make_async_copy(
                    ttn_hbm.at[:, pl.ds(0, 128)], s_v, isem).wait()

                @pl.when(j >= 1)
                def _reclaim_t():
                    pltpu.make_async_copy(
                        t_v, tab4_hbm.at[pl.ds(0, 32)], osem).wait()

                transpose(s_v, t_v, 32)
                pltpu.async_copy(
                    t_v, tab4_hbm.at[pl.ds(c * 32, 32)], osem)

                @pl.when(t + 2 < ntrips)
                def _next_in():
                    cn = wid + (t + 2) * NW
                    pltpu.async_copy(
                        ttn_hbm.at[:, pl.ds(cn * 128, 128)], s_v, isem)

        return carry

    lax.fori_loop(0, (245 + 1) // 2, trip_pair, 0)

    for _, t_v, _, osem in bufs:
        pltpu.make_async_copy(t_v, tab4_hbm.at[pl.ds(0, 32)], osem).wait()

    @pl.when(wid == NW - 1)
    def _tail():
        pltpu.sync_copy(tail4_hbm, tab4_hbm.at[pl.ds(NFULL * 32, 16)])


@functools.partial(
    pl.kernel,
    mesh=_mesh,
    out_type=jax.ShapeDtypeStruct((HIST, N_EMBED, BATCH), jnp.float32),
    scratch_types=[
        pltpu.VMEM((HIST, NB), jnp.int32),
        pltpu.VMEM((HIST, NB), jnp.int32),
        pltpu.VMEM((HIST, NB), jnp.int32),
        pltpu.VMEM((NB, 128), jnp.float32),
        pltpu.VMEM((NB, 128), jnp.float32),
        pltpu.VMEM((N_EMBED, NB), jnp.float32),
        pltpu.VMEM((N_EMBED, NB), jnp.float32),
        pltpu.SemaphoreType.DMA,
        pltpu.SemaphoreType.DMA,
        pltpu.SemaphoreType.DMA,
        pltpu.SemaphoreType.DMA,
    ],
    compiler_params=pltpu.CompilerParams(
        use_tc_tiling_on_sc=True, needs_layout_passes=False),
)
def _gather_kernel(xt_hbm, tab4_hbm, out_hbm, xt_v, idx4_v, rem32_v,
                   g0, g1, d0, d1, gsem0, gsem1, ssem0, ssem1):
    wid = lax.axis_index("s") * NC + lax.axis_index("c")
    nbase = wid * N_PER_W
    bufs = ((g0, d0, gsem0, ssem0), (g1, d1, gsem1, ssem1))
    lanes = lax.iota(jnp.int32, 16)

    def extract(g_v, d_v, h):
        for g in range(8):
            lvec = lanes + (16 * g)
            cols0 = rem32_v[h, pl.ds(16 * g, 16)]

            def e_body(e, carry):
                vals = plsc.load_gather(g_v, [lvec, cols0 + e])
                d_v[e, pl.ds(16 * g, 16)] = vals
                return carry

            lax.fori_loop(0, N_EMBED, e_body, 0)

    def nblk_body(nblk, carry):
        n0 = nbase + nblk * NB
        pltpu.sync_copy(xt_hbm.at[:, pl.ds(n0, NB)], xt_v)
        def prep_body(h, carry):
            for g in range(8):
                v = xt_v[h, pl.ds(16 * g, 16)]
                idx4_v[h, pl.ds(16 * g, 16)] = v >> 2
                rem32_v[h, pl.ds(16 * g, 16)] = (v & 3) << 5
            return carry

        lax.fori_loop(0, HIST, prep_body, 0)

        pltpu.async_copy(tab4_hbm.at[idx4_v.at[0]], g0, gsem0)
        pltpu.async_copy(tab4_hbm.at[idx4_v.at[1]], g1, gsem1)

        def pair_body(j, carry):
            for b, (g_v, d_v, gsem, ssem) in enumerate(bufs):
                h = 2 * j + b

                @pl.when(j >= 1)
                def _reclaim_d():
                    pltpu.make_async_copy(
                        d_v, out_hbm.at[0, :, pl.ds(n0, NB)], ssem).wait()

                pltpu.make_async_copy(
                    tab4_hbm.at[idx4_v.at[h]], g_v, gsem).wait()
                extract(g_v, d_v, h)
                pltpu.async_copy(
                    d_v, out_hbm.at[h, :, pl.ds(n0, NB)], ssem)

                @pl.when(h + 2 < HIST)
                def _next_gather():
                    pltpu.async_copy(
                        tab4_hbm.at[idx4_v.at[h + 2]], g_v, gsem)
            return carry

        lax.fori_loop(0, NPAIR, pair_body, 0)
        for _, d_v, _, ssem in bufs:
            pltpu.make_async_copy(
                d_v, out_hbm.at[0, :, pl.ds(n0, NB)], ssem).wait()
        return carry

    lax.fori_loop(0, NBLK, nblk_body, 0)


def kernel(x, table):
    xt = x.T.astype(jnp.int32)
    tail4 = lax.slice(table, (NUM_ACTIONS - 64, 0), (NUM_ACTIONS, N_EMBED))
    tab4 = _detile_kernel(table.T, tail4.reshape(16, 128))
    out_t = _gather_kernel(xt, tab4)
    return jnp.transpose(out_t, (2, 0, 1))


# speedup vs baseline: 1.2373x; 1.2373x over previous
import functools

import jax
import jax.numpy as jnp
from jax import lax
from jax.experimental import pallas as pl
from jax.experimental.pallas import tpu as pltpu
from jax.experimental.pallas import tpu_sc as plsc

NUM_ACTIONS = 1000000
N_EMBED = 32
BATCH = 16384
HIST = 50

NC, NS = 2, 16
NW = NC * NS
N_PER_W = BATCH // NW
NB = 128
NBLK = N_PER_W // NB
NPAIR = HIST // 2

_mesh = plsc.VectorSubcoreMesh(core_axis_name="c", subcore_axis_name="s")


@functools.partial(
    pl.kernel,
    mesh=_mesh,
    out_type=jax.ShapeDtypeStruct((HIST, N_EMBED, BATCH), jnp.float32),
    scratch_types=[
        pltpu.VMEM((HIST, NB), jnp.int32),
        pltpu.VMEM((HIST, NB), jnp.int32),
        pltpu.VMEM((HIST, NB), jnp.int32),
        pltpu.VMEM((NB, 128), jnp.float32),
        pltpu.VMEM((NB, 128), jnp.float32),
        pltpu.VMEM((N_EMBED, NB), jnp.float32),
        pltpu.VMEM((N_EMBED, NB), jnp.float32),
        pltpu.SemaphoreType.DMA,
        pltpu.SemaphoreType.DMA,
        pltpu.SemaphoreType.DMA,
        pltpu.SemaphoreType.DMA,
    ],
    compiler_params=pltpu.CompilerParams(
        use_tc_tiling_on_sc=True, needs_layout_passes=False),
)
def _gather_kernel(xt_hbm, tab4_hbm, out_hbm, xt_v, idx4_v, rem32_v,
                   g0, g1, d0, d1, gsem0, gsem1, ssem0, ssem1):
    wid = lax.axis_index("s") * NC + lax.axis_index("c")
    nbase = wid * N_PER_W
    bufs = ((g0, d0, gsem0, ssem0), (g1, d1, gsem1, ssem1))
    lanes = lax.iota(jnp.int32, 16)

    def extract(g_v, d_v, h):
        for g in range(8):
            lvec = lanes + (16 * g)
            cols0 = rem32_v[h, pl.ds(16 * g, 16)]

            def e_body(e, carry):
                vals = plsc.load_gather(g_v, [lvec, cols0 + e])
                d_v[e, pl.ds(16 * g, 16)] = vals
                return carry

            lax.fori_loop(0, N_EMBED, e_body, 0)

    def nblk_body(nblk, carry):
        n0 = nbase + nblk * NB
        pltpu.sync_copy(xt_hbm.at[:, pl.ds(n0, NB)], xt_v)
        def prep_body(h, carry):
            for g in range(8):
                v = xt_v[h, pl.ds(16 * g, 16)]
                idx4_v[h, pl.ds(16 * g, 16)] = v >> 2
                rem32_v[h, pl.ds(16 * g, 16)] = (v & 3) << 5
            return carry

        lax.fori_loop(0, HIST, prep_body, 0)

        pltpu.async_copy(tab4_hbm.at[idx4_v.at[0]], g0, gsem0)
        pltpu.async_copy(tab4_hbm.at[idx4_v.at[1]], g1, gsem1)

        def pair_body(j, carry):
            for b, (g_v, d_v, gsem, ssem) in enumerate(bufs):
                h = 2 * j + b

                @pl.when(j >= 1)
                def _reclaim_d():
                    pltpu.make_async_copy(
                        d_v, out_hbm.at[0, :, pl.ds(n0, NB)], ssem).wait()

                pltpu.make_async_copy(
                    tab4_hbm.at[idx4_v.at[h]], g_v, gsem).wait()
                extract(g_v, d_v, h)
                pltpu.async_copy(
                    d_v, out_hbm.at[h, :, pl.ds(n0, NB)], ssem)

                @pl.when(h + 2 < HIST)
                def _next_gather():
                    pltpu.async_copy(
                        tab4_hbm.at[idx4_v.at[h + 2]], g_v, gsem)
            return carry

        lax.fori_loop(0, NPAIR, pair_body, 0)
        for _, d_v, _, ssem in bufs:
            pltpu.make_async_copy(
                d_v, out_hbm.at[0, :, pl.ds(n0, NB)], ssem).wait()
        return carry

    lax.fori_loop(0, NBLK, nblk_body, 0)


def kernel(x, table):
    xt = x.T.astype(jnp.int32)
    tab4 = table.reshape(NUM_ACTIONS // 4, 128)
    out_t = _gather_kernel(xt, tab4)
    return jnp.transpose(out_t, (2, 0, 1))
